# baseline (speedup 1.0000x reference)
import jax
import jax.numpy as jnp
from jax.experimental import pallas as pl
from jax.experimental.pallas import tpu as pltpu

_TN = 512
_KA = 128


def _chamfer_kernel(p_ref, t_ref, loss_ref, aa_ref, bb_ref, runmin_ref):
    b = pl.program_id(0)
    n = pl.program_id(1)
    nt = pl.num_programs(1)
    n_rows = p_ref.shape[1]
    m = t_ref.shape[1]
    d = p_ref.shape[2]
    inv_bn = 1.0 / (pl.num_programs(0) * n_rows)
    inv_bm = 1.0 / (pl.num_programs(0) * m)

    @pl.when(n == 0)
    def _():
        p = p_ref[0]
        t = t_ref[0]
        p2 = jnp.sum(p * p, axis=1, keepdims=True)
        t2 = jnp.sum(t * t, axis=1, keepdims=True)
        aa_ref[...] = jnp.concatenate(
            [(p * -2.0).astype(jnp.bfloat16), p2.astype(jnp.bfloat16),
             jnp.ones((n_rows, 1), jnp.bfloat16),
             jnp.zeros((n_rows, _KA - d - 2), jnp.bfloat16)], axis=1)
        bb_ref[...] = jnp.concatenate(
            [t.astype(jnp.bfloat16), jnp.ones((m, 1), jnp.bfloat16),
             t2.astype(jnp.bfloat16),
             jnp.zeros((m, _KA - d - 2), jnp.bfloat16)], axis=1)

    a_blk = aa_ref[pl.ds(n * _TN, _TN), :]
    dist = jax.lax.dot_general(
        a_blk, bb_ref[...], (((1,), (1,)), ((), ())),
        preferred_element_type=jnp.float32)

    cham_x = jnp.maximum(jnp.min(dist, axis=1, keepdims=True), 0.0)

    @pl.when(jnp.logical_and(b == 0, n == 0))
    def _():
        loss_ref[0, 0] = 0.0

    loss_ref[0, 0] += jnp.sum(cham_x) * inv_bn

    col_min = jnp.min(dist, axis=0, keepdims=True)

    @pl.when(n == 0)
    def _():
        runmin_ref[...] = col_min

    @pl.when(n != 0)
    def _():
        runmin_ref[...] = jnp.minimum(runmin_ref[...], col_min)

    @pl.when(n == nt - 1)
    def _():
        cham_y = jnp.maximum(runmin_ref[...], 0.0)
        loss_ref[0, 0] += jnp.sum(cham_y) * inv_bm


def kernel(pred, target):
    bsz, n, d = pred.shape
    m = target.shape[1]
    out = pl.pallas_call(
        _chamfer_kernel,
        grid=(bsz, n // _TN),
        in_specs=[
            pl.BlockSpec((1, n, d), lambda b, i: (b, 0, 0)),
            pl.BlockSpec((1, m, d), lambda b, i: (b, 0, 0)),
        ],
        out_specs=pl.BlockSpec(
            (1, 1), lambda b, i: (0, 0), memory_space=pltpu.SMEM),
        out_shape=jax.ShapeDtypeStruct((1, 1), jnp.float32),
        scratch_shapes=[
            pltpu.VMEM((n, _KA), jnp.bfloat16),
            pltpu.VMEM((m, _KA), jnp.bfloat16),
            pltpu.VMEM((1, m), jnp.float32),
        ],
        compiler_params=pltpu.CompilerParams(
            dimension_semantics=("arbitrary", "arbitrary")),
    )(pred, target)
    return out[0, 0]


# speedup vs baseline: 1.0894x; 1.0894x over previous
import jax
import jax.numpy as jnp
from jax.experimental import pallas as pl
from jax.experimental.pallas import tpu as pltpu

_TN = 512


def _chamfer_kernel(p_ref, t_ref, loss_ref, cm_ref, rp_ref):
    b = pl.program_id(0)
    n = pl.program_id(1)
    nt = pl.num_programs(1)
    m = t_ref.shape[1]
    n_rows = nt * _TN
    inv_bn = 1.0 / (pl.num_programs(0) * n_rows)
    inv_bm = 1.0 / (pl.num_programs(0) * m)

    p = p_ref[0]
    t = t_ref[0]
    p2 = jnp.sum(p * p, axis=1, keepdims=True)
    t2 = jnp.sum(t * t, axis=1, keepdims=True)

    a_aug = jnp.concatenate(
        [(p * -2.0).astype(jnp.bfloat16), p2.astype(jnp.bfloat16),
         jnp.ones((_TN, 1), jnp.bfloat16)], axis=1)
    b_aug = jnp.concatenate(
        [t.astype(jnp.bfloat16), jnp.ones((m, 1), jnp.bfloat16),
         t2.astype(jnp.bfloat16)], axis=1)
    dist = jax.lax.dot_general(
        a_aug, b_aug, (((1,), (1,)), ((), ())),
        preferred_element_type=jnp.float32)

    rm = dist[:, 0:128]
    for i in range(1, m // 128):
        rm = jnp.minimum(rm, dist[:, i * 128:(i + 1) * 128])
    rp_ref[pl.ds(n * _TN, _TN), :] = rm

    cm = dist[0:8, :]
    for k in range(1, _TN // 8):
        cm = jnp.minimum(cm, dist[k * 8:(k + 1) * 8, :])

    @pl.when(n == 0)
    def _():
        cm_ref[...] = cm

    @pl.when(n != 0)
    def _():
        cm_ref[...] = jnp.minimum(cm_ref[...], cm)

    @pl.when(jnp.logical_and(b == 0, n == 0))
    def _():
        loss_ref[0, 0] = 0.0

    @pl.when(n == nt - 1)
    def _():
        cham_x = jnp.maximum(
            jnp.min(rp_ref[...], axis=1, keepdims=True), 0.0)
        cham_y = jnp.maximum(
            jnp.min(cm_ref[...], axis=0, keepdims=True), 0.0)
        loss_ref[0, 0] += (jnp.sum(cham_x) * inv_bn
                           + jnp.sum(cham_y) * inv_bm)


def kernel(pred, target):
    bsz, n, d = pred.shape
    m = target.shape[1]
    out = pl.pallas_call(
        _chamfer_kernel,
        grid=(bsz, n // _TN),
        in_specs=[
            pl.BlockSpec((1, _TN, d), lambda b, i: (b, i, 0)),
            pl.BlockSpec((1, m, d), lambda b, i: (b, 0, 0)),
        ],
        out_specs=pl.BlockSpec(
            (1, 1), lambda b, i: (0, 0), memory_space=pltpu.SMEM),
        out_shape=jax.ShapeDtypeStruct((1, 1), jnp.float32),
        scratch_shapes=[
            pltpu.VMEM((8, m), jnp.float32),
            pltpu.VMEM((n, 128), jnp.float32),
        ],
        compiler_params=pltpu.CompilerParams(
            dimension_semantics=("arbitrary", "arbitrary")),
    )(pred, target)
    return out[0, 0]


# speedup vs baseline: 1.2109x; 1.1115x over previous
import jax
import jax.numpy as jnp
from jax.experimental import pallas as pl
from jax.experimental.pallas import tpu as pltpu

_TN = 1024


def _chamfer_kernel(p_ref, t_ref, loss_ref, cm_ref, rp_ref):
    b = pl.program_id(0)
    n = pl.program_id(1)
    nt = pl.num_programs(1)
    m = t_ref.shape[1]
    n_rows = nt * _TN
    inv_bn = 1.0 / (pl.num_programs(0) * n_rows)
    inv_bm = 1.0 / (pl.num_programs(0) * m)

    p = p_ref[0]
    t = t_ref[0]
    p2 = jnp.sum(p * p, axis=1, keepdims=True)
    t2 = jnp.sum(t * t, axis=1, keepdims=True)

    a_aug = jnp.concatenate(
        [(p * -2.0).astype(jnp.bfloat16), p2.astype(jnp.bfloat16),
         jnp.ones((_TN, 1), jnp.bfloat16)], axis=1)
    b_aug = jnp.concatenate(
        [t.astype(jnp.bfloat16), jnp.ones((m, 1), jnp.bfloat16),
         t2.astype(jnp.bfloat16)], axis=1)
    dist = jax.lax.dot_general(
        a_aug, b_aug, (((1,), (1,)), ((), ())),
        preferred_element_type=jnp.float32)

    rm = dist[:, 0:128]
    for i in range(1, m // 128):
        rm = jnp.minimum(rm, dist[:, i * 128:(i + 1) * 128])
    rp_ref[pl.ds(n * _TN, _TN), :] = rm

    cm = dist[0:8, :]
    for k in range(1, _TN // 8):
        cm = jnp.minimum(cm, dist[k * 8:(k + 1) * 8, :])

    @pl.when(n == 0)
    def _():
        cm_ref[...] = cm

    @pl.when(n != 0)
    def _():
        cm_ref[...] = jnp.minimum(cm_ref[...], cm)

    @pl.when(jnp.logical_and(b == 0, n == 0))
    def _():
        loss_ref[0, 0] = 0.0

    @pl.when(n == nt - 1)
    def _():
        cham_x = jnp.maximum(
            jnp.min(rp_ref[...], axis=1, keepdims=True), 0.0)
        cham_y = jnp.maximum(
            jnp.min(cm_ref[...], axis=0, keepdims=True), 0.0)
        loss_ref[0, 0] += (jnp.sum(cham_x) * inv_bn
                           + jnp.sum(cham_y) * inv_bm)


def kernel(pred, target):
    bsz, n, d = pred.shape
    m = target.shape[1]
    out = pl.pallas_call(
        _chamfer_kernel,
        grid=(bsz, n // _TN),
        in_specs=[
            pl.BlockSpec((1, _TN, d), lambda b, i: (b, i, 0)),
            pl.BlockSpec((1, m, d), lambda b, i: (b, 0, 0)),
        ],
        out_specs=pl.BlockSpec(
            (1, 1), lambda b, i: (0, 0), memory_space=pltpu.SMEM),
        out_shape=jax.ShapeDtypeStruct((1, 1), jnp.float32),
        scratch_shapes=[
            pltpu.VMEM((8, m), jnp.float32),
            pltpu.VMEM((n, 128), jnp.float32),
        ],
        compiler_params=pltpu.CompilerParams(
            dimension_semantics=("arbitrary", "arbitrary")),
    )(pred, target)
    return out[0, 0]


# speedup vs baseline: 1.2741x; 1.0522x over previous
import jax
import jax.numpy as jnp
from jax.experimental import pallas as pl
from jax.experimental.pallas import tpu as pltpu

_TN = 2048


def _chamfer_kernel(p_ref, t_ref, loss_ref, cm_ref, rp_ref):
    b = pl.program_id(0)
    n = pl.program_id(1)
    nt = pl.num_programs(1)
    m = t_ref.shape[1]
    n_rows = nt * _TN
    inv_bn = 1.0 / (pl.num_programs(0) * n_rows)
    inv_bm = 1.0 / (pl.num_programs(0) * m)

    p = p_ref[0]
    t = t_ref[0]
    p2 = jnp.sum(p * p, axis=1, keepdims=True)
    t2 = jnp.sum(t * t, axis=1, keepdims=True)

    a_aug = jnp.concatenate(
        [(p * -2.0).astype(jnp.bfloat16), p2.astype(jnp.bfloat16),
         jnp.ones((_TN, 1), jnp.bfloat16)], axis=1)
    b_aug = jnp.concatenate(
        [t.astype(jnp.bfloat16), jnp.ones((m, 1), jnp.bfloat16),
         t2.astype(jnp.bfloat16)], axis=1)
    dist = jax.lax.dot_general(
        a_aug, b_aug, (((1,), (1,)), ((), ())),
        preferred_element_type=jnp.float32)

    rm = dist[:, 0:128]
    for i in range(1, m // 128):
        rm = jnp.minimum(rm, dist[:, i * 128:(i + 1) * 128])
    rp_ref[pl.ds(n * _TN, _TN), :] = rm

    cm = dist[0:8, :]
    for k in range(1, _TN // 8):
        cm = jnp.minimum(cm, dist[k * 8:(k + 1) * 8, :])

    @pl.when(n == 0)
    def _():
        cm_ref[...] = cm

    @pl.when(n != 0)
    def _():
        cm_ref[...] = jnp.minimum(cm_ref[...], cm)

    @pl.when(jnp.logical_and(b == 0, n == 0))
    def _():
        loss_ref[0, 0] = 0.0

    @pl.when(n == nt - 1)
    def _():
        cham_x = jnp.maximum(
            jnp.min(rp_ref[...], axis=1, keepdims=True), 0.0)
        cham_y = jnp.maximum(
            jnp.min(cm_ref[...], axis=0, keepdims=True), 0.0)
        loss_ref[0, 0] += (jnp.sum(cham_x) * inv_bn
                           + jnp.sum(cham_y) * inv_bm)


def kernel(pred, target):
    bsz, n, d = pred.shape
    m = target.shape[1]
    out = pl.pallas_call(
        _chamfer_kernel,
        grid=(bsz, n // _TN),
        in_specs=[
            pl.BlockSpec((1, _TN, d), lambda b, i: (b, i, 0)),
            pl.BlockSpec((1, m, d), lambda b, i: (b, 0, 0)),
        ],
        out_specs=pl.BlockSpec(
            (1, 1), lambda b, i: (0, 0), memory_space=pltpu.SMEM),
        out_shape=jax.ShapeDtypeStruct((1, 1), jnp.float32),
        scratch_shapes=[
            pltpu.VMEM((8, m), jnp.float32),
            pltpu.VMEM((n, 128), jnp.float32),
        ],
        compiler_params=pltpu.CompilerParams(
            dimension_semantics=("arbitrary", "arbitrary")),
    )(pred, target)
    return out[0, 0]


# speedup vs baseline: 1.2779x; 1.0030x over previous
import jax
import jax.numpy as jnp
from jax.experimental import pallas as pl
from jax.experimental.pallas import tpu as pltpu

_TN = 2048


def _chamfer_kernel(p_ref, t_ref, loss_ref, cm_ref, rp_ref):
    b = pl.program_id(0)
    nb = pl.num_programs(0)
    n = pl.program_id(1)
    nt = pl.num_programs(1)
    m = t_ref.shape[1]
    n_rows = nt * _TN
    inv_bn = 1.0 / (nb * n_rows)
    inv_bm = 1.0 / (nb * m)
    par = jax.lax.rem(b, 2)

    p = p_ref[0]
    t = t_ref[0]
    p2 = jnp.sum(p * p, axis=1, keepdims=True)
    t2 = jnp.sum(t * t, axis=1, keepdims=True)

    a_aug = jnp.concatenate(
        [(p * -2.0).astype(jnp.bfloat16), p2.astype(jnp.bfloat16),
         jnp.ones((_TN, 1), jnp.bfloat16)], axis=1)
    b_aug = jnp.concatenate(
        [t.astype(jnp.bfloat16), jnp.ones((m, 1), jnp.bfloat16),
         t2.astype(jnp.bfloat16)], axis=1)
    dist = jax.lax.dot_general(
        a_aug, b_aug, (((1,), (1,)), ((), ())),
        preferred_element_type=jnp.float32)

    rm = dist[:, 0:128]
    for i in range(1, m // 128):
        rm = jnp.minimum(rm, dist[:, i * 128:(i + 1) * 128])
    rp_ref[pl.ds(par * n_rows + n * _TN, _TN), :] = rm

    cm = dist[0:8, :]
    for k in range(1, _TN // 8):
        cm = jnp.minimum(cm, dist[k * 8:(k + 1) * 8, :])

    @pl.when(n == 0)
    def _():
        cm_ref[pl.ds(par * 8, 8), :] = cm

    @pl.when(n != 0)
    def _():
        cm_ref[pl.ds(par * 8, 8), :] = jnp.minimum(
            cm_ref[pl.ds(par * 8, 8), :], cm)

    @pl.when(jnp.logical_and(b == 0, n == 0))
    def _():
        loss_ref[0, 0] = 0.0

    def _finalize(buf):
        rp = rp_ref[pl.ds(buf * n_rows, n_rows), :]
        cham_x = jnp.maximum(jnp.min(rp, axis=1, keepdims=True), 0.0)
        cmb = cm_ref[pl.ds(buf * 8, 8), :]
        cham_y = jnp.maximum(jnp.min(cmb, axis=0, keepdims=True), 0.0)
        loss_ref[0, 0] += (jnp.sum(cham_x) * inv_bn
                           + jnp.sum(cham_y) * inv_bm)

    @pl.when(jnp.logical_and(b > 0, n == 0))
    def _():
        _finalize(1 - par)

    @pl.when(jnp.logical_and(b == nb - 1, n == nt - 1))
    def _():
        _finalize(par)


def kernel(pred, target):
    bsz, n, d = pred.shape
    m = target.shape[1]
    out = pl.pallas_call(
        _chamfer_kernel,
        grid=(bsz, n // _TN),
        in_specs=[
            pl.BlockSpec((1, _TN, d), lambda b, i: (b, i, 0)),
            pl.BlockSpec((1, m, d), lambda b, i: (b, 0, 0)),
        ],
        out_specs=pl.BlockSpec(
            (1, 1), lambda b, i: (0, 0), memory_space=pltpu.SMEM),
        out_shape=jax.ShapeDtypeStruct((1, 1), jnp.float32),
        scratch_shapes=[
            pltpu.VMEM((16, m), jnp.float32),
            pltpu.VMEM((2 * n, 128), jnp.float32),
        ],
        compiler_params=pltpu.CompilerParams(
            dimension_semantics=("arbitrary", "arbitrary")),
    )(pred, target)
    return out[0, 0]
